# Initial kernel scaffold; baseline (speedup 1.0000x reference)
import jax
import jax.numpy as jnp
from jax.experimental import pallas as pl


def kernel(feat, h, edge_index, W_val, b_val, W_sign, b_sign, W_que, b_que, W_ih, b_ih, W_hh, b_hh):
    raise NotImplementedError("write your pallas kernel here")


# speedup vs baseline: 1.8362x; 1.8362x over previous
import functools

import jax
import jax.numpy as jnp
from jax import lax
from jax.experimental import pallas as pl
from jax.experimental.pallas import tpu as pltpu

N = 10000
E = 160000
H = 256
MSG = 64
KEY = 32

NPAD = 10016
ACC_W = 80

BN = 1000
GRID = N // BN


def _proj0_body(feat, h, wft, wht, bp, bih, bhh,
                pf_o, gf_o, v_o, s_o, q_o, gh_o):
    a = jnp.dot(feat[...], wft[...], preferred_element_type=jnp.float32)
    b = jnp.dot(h[...], wht[...], preferred_element_type=jnp.float32)
    pf = a[:, :128] + bp[...]
    gf = a[:, 128:] + bih[...]
    p1 = pf + b[:, :128]
    pf_o[...] = pf
    gf_o[...] = gf
    v_o[...] = p1[:, :MSG]
    s_o[...] = p1[:, MSG:MSG + KEY]
    q_o[...] = p1[:, MSG + KEY:]
    gh_o[...] = b[:, 128:] + bhh[...]


def _gru_core(c, gf, gh, h, wihct):
    gi = gf + jnp.dot(c, wihct, preferred_element_type=jnp.float32)
    i_r, i_z, i_n = gi[:, :H], gi[:, H:2 * H], gi[:, 2 * H:]
    h_r, h_z, h_n = gh[:, :H], gh[:, H:2 * H], gh[:, 2 * H:]
    r = jax.nn.sigmoid(i_r + h_r)
    z = jax.nn.sigmoid(i_z + h_z)
    n = jnp.tanh(i_n + r * h_n)
    return (1.0 - z) * n + z * h


def _finalize_c(acc):
    a = jnp.sum(acc[...], axis=0)
    den = jnp.sum(a[:, :16], axis=1)
    num = a[:, 16:]
    return num * (1.0 / jnp.maximum(den, 1e-30))[:, None]


def _gru_proj_body(acc, gf, gh, h, pf, wihct, wht, bhh,
                   h1_o, v_o, s_o, q_o, gh_o):
    c = _finalize_c(acc)
    h1 = _gru_core(c, gf[...], gh[...], h[...], wihct[...])
    h1_o[...] = h1
    b2 = jnp.dot(h1, wht[...], preferred_element_type=jnp.float32)
    p2 = pf[...] + b2[:, :128]
    v_o[...] = p2[:, :MSG]
    s_o[...] = p2[:, MSG:MSG + KEY]
    q_o[...] = p2[:, MSG + KEY:]
    gh_o[...] = b2[:, 128:] + bhh[...]


def _gru_final_body(acc, gf, gh, h, wihct, h2_o):
    c = _finalize_c(acc)
    h2_o[...] = _gru_core(c, gf[...], gh[...], h[...], wihct[...])


def _row_spec(w):
    return pl.BlockSpec((BN, w), lambda i: (i, 0))


def _full_spec(shape):
    return pl.BlockSpec(shape, lambda i: tuple(0 for _ in shape))


def _proj0(feat, h, wft, wht, bp, bih, bhh):
    return pl.pallas_call(
        _proj0_body,
        grid=(GRID,),
        in_specs=[_row_spec(H), _row_spec(H), _full_spec((H, 896)),
                  _full_spec((H, 896)), _full_spec((1, 128)),
                  _full_spec((1, 768)), _full_spec((1, 768))],
        out_specs=[_row_spec(128), _row_spec(768), _row_spec(MSG),
                   _row_spec(KEY), _row_spec(KEY), _row_spec(768)],
        out_shape=[jax.ShapeDtypeStruct((N, 128), jnp.float32),
                   jax.ShapeDtypeStruct((N, 768), jnp.float32),
                   jax.ShapeDtypeStruct((N, MSG), jnp.float32),
                   jax.ShapeDtypeStruct((N, KEY), jnp.float32),
                   jax.ShapeDtypeStruct((N, KEY), jnp.float32),
                   jax.ShapeDtypeStruct((N, 768), jnp.float32)],
        compiler_params=pltpu.CompilerParams(
            dimension_semantics=("parallel",)),
    )(feat, h, wft, wht, bp, bih, bhh)


def _acc_spec(nacc):
    return pl.BlockSpec((nacc, BN, ACC_W), lambda i: (0, i, 0))


def _gru_proj(acc, gf, gh, h, pf, wihct, wht, bhh):
    return pl.pallas_call(
        _gru_proj_body,
        grid=(GRID,),
        in_specs=[_acc_spec(acc.shape[0]),
                  _row_spec(768), _row_spec(768), _row_spec(H),
                  _row_spec(128), _full_spec((MSG, 768)),
                  _full_spec((H, 896)), _full_spec((1, 768))],
        out_specs=[_row_spec(H), _row_spec(MSG), _row_spec(KEY),
                   _row_spec(KEY), _row_spec(768)],
        out_shape=[jax.ShapeDtypeStruct((N, H), jnp.float32),
                   jax.ShapeDtypeStruct((N, MSG), jnp.float32),
                   jax.ShapeDtypeStruct((N, KEY), jnp.float32),
                   jax.ShapeDtypeStruct((N, KEY), jnp.float32),
                   jax.ShapeDtypeStruct((N, 768), jnp.float32)],
        compiler_params=pltpu.CompilerParams(
            dimension_semantics=("parallel",)),
    )(acc, gf, gh, h, pf, wihct, wht, bhh)


def _gru_final(acc, gf, gh, h, wihct):
    return pl.pallas_call(
        _gru_final_body,
        grid=(GRID,),
        in_specs=[_acc_spec(acc.shape[0]),
                  _row_spec(768), _row_spec(768), _row_spec(H),
                  _full_spec((MSG, 768))],
        out_specs=_row_spec(H),
        out_shape=jax.ShapeDtypeStruct((N, H), jnp.float32),
        compiler_params=pltpu.CompilerParams(
            dimension_semantics=("parallel",)),
    )(acc, gf, gh, h, wihct)


def _edge_pass(src, dst, s, q, v):
    e = jnp.sum(s[src] * q[dst], axis=1) * (1.0 / KEY)
    ee = jnp.exp(e)
    den = jax.ops.segment_sum(ee, dst, num_segments=NPAD)
    num = jax.ops.segment_sum(v[src] * ee[:, None], dst, num_segments=NPAD)
    acc = jnp.zeros((NPAD, ACC_W), jnp.float32)
    acc = acc.at[:, 0].set(den).at[:, 16:].set(num)
    return acc[None]


def kernel(feat, h, edge_index, W_val, b_val, W_sign, b_sign, W_que, b_que,
           W_ih, b_ih, W_hh, b_hh):
    src = edge_index[0]
    dst = edge_index[1]
    wf = jnp.concatenate([W_val[:, :H], W_sign[:, :H], W_que[:, :H],
                          W_ih[:, :H]], axis=0)
    wh = jnp.concatenate([W_val[:, H:], W_sign[:, H:], W_que[:, H:],
                          W_hh], axis=0)
    wft = wf.T
    wht = wh.T
    wihct = W_ih[:, H:].T
    bp = jnp.concatenate([b_val, b_sign, b_que])[None]
    bih = b_ih[None]
    bhh = b_hh[None]

    pf, gf, v1, s1, q1, gh1 = _proj0(feat, h, wft, wht, bp, bih, bhh)
    acc1 = _edge_pass(src, dst, s1, q1, v1)
    h1, v2, s2, q2, gh2 = _gru_proj(acc1, gf, gh1, h, pf, wihct, wht, bhh)
    acc2 = _edge_pass(src, dst, s2, q2, v2)
    h2 = _gru_final(acc2, gf, gh2, h1, wihct)
    return (h2, h2)


# speedup vs baseline: 3.8558x; 2.0999x over previous
import functools

import jax
import jax.numpy as jnp
from jax import lax
from jax.experimental import pallas as pl
from jax.experimental.pallas import tpu as pltpu
from jax.experimental.pallas import tpu_sc as plsc

N = 10000
E = 160000
H = 256
MSG = 64
KEY = 32

NPAD = 10112
ACC_W = 80

BN = 1000
GRID = N // BN


def _proj0_body(feat, h, wft, wht, bp, bih, bhh,
                pf_o, gf_o, v_o, s_o, q_o, gh_o):
    a = jnp.dot(feat[...], wft[...], preferred_element_type=jnp.float32)
    b = jnp.dot(h[...], wht[...], preferred_element_type=jnp.float32)
    pf = a[:, :128] + bp[...]
    gf = a[:, 128:] + bih[...]
    p1 = pf + b[:, :128]
    pf_o[...] = pf
    gf_o[...] = gf
    v_o[...] = p1[:, :MSG]
    s_o[...] = p1[:, MSG:MSG + KEY]
    q_o[...] = p1[:, MSG + KEY:]
    gh_o[...] = b[:, 128:] + bhh[...]


def _gru_core(c, gf, gh, h, wihct):
    gi = gf + jnp.dot(c, wihct, preferred_element_type=jnp.float32)
    i_r, i_z, i_n = gi[:, :H], gi[:, H:2 * H], gi[:, 2 * H:]
    h_r, h_z, h_n = gh[:, :H], gh[:, H:2 * H], gh[:, 2 * H:]
    r = jax.nn.sigmoid(i_r + h_r)
    z = jax.nn.sigmoid(i_z + h_z)
    n = jnp.tanh(i_n + r * h_n)
    return (1.0 - z) * n + z * h


def _finalize_c(acc):
    a = jnp.sum(acc[...], axis=0)
    den = jnp.sum(a[:, :16], axis=1)
    num = a[:, 16:]
    return num * (1.0 / jnp.maximum(den, 1e-30))[:, None]


def _gru_proj_body(acc, gf, gh, h, pf, wihct, wht, bhh,
                   h1_o, v_o, s_o, q_o, gh_o):
    c = _finalize_c(acc)
    h1 = _gru_core(c, gf[...], gh[...], h[...], wihct[...])
    h1_o[...] = h1
    b2 = jnp.dot(h1, wht[...], preferred_element_type=jnp.float32)
    p2 = pf[...] + b2[:, :128]
    v_o[...] = p2[:, :MSG]
    s_o[...] = p2[:, MSG:MSG + KEY]
    q_o[...] = p2[:, MSG + KEY:]
    gh_o[...] = b2[:, 128:] + bhh[...]


def _gru_final_body(acc, gf, gh, h, wihct, h2_o):
    c = _finalize_c(acc)
    h2_o[...] = _gru_core(c, gf[...], gh[...], h[...], wihct[...])


def _row_spec(w):
    return pl.BlockSpec((BN, w), lambda i: (i, 0))


def _full_spec(shape):
    return pl.BlockSpec(shape, lambda i: tuple(0 for _ in shape))


def _proj0(feat, h, wft, wht, bp, bih, bhh):
    return pl.pallas_call(
        _proj0_body,
        grid=(GRID,),
        in_specs=[_row_spec(H), _row_spec(H), _full_spec((H, 896)),
                  _full_spec((H, 896)), _full_spec((1, 128)),
                  _full_spec((1, 768)), _full_spec((1, 768))],
        out_specs=[_row_spec(128), _row_spec(768), _row_spec(MSG),
                   _row_spec(KEY), _row_spec(KEY), _row_spec(768)],
        out_shape=[jax.ShapeDtypeStruct((N, 128), jnp.float32),
                   jax.ShapeDtypeStruct((N, 768), jnp.float32),
                   jax.ShapeDtypeStruct((N, MSG), jnp.float32),
                   jax.ShapeDtypeStruct((N, KEY), jnp.float32),
                   jax.ShapeDtypeStruct((N, KEY), jnp.float32),
                   jax.ShapeDtypeStruct((N, 768), jnp.float32)],
        compiler_params=pltpu.CompilerParams(
            dimension_semantics=("parallel",)),
    )(feat, h, wft, wht, bp, bih, bhh)


def _acc_spec(nacc):
    return pl.BlockSpec((nacc, BN, ACC_W), lambda i: (0, i, 0))


def _gru_proj(acc, gf, gh, h, pf, wihct, wht, bhh):
    return pl.pallas_call(
        _gru_proj_body,
        grid=(GRID,),
        in_specs=[_acc_spec(acc.shape[0]),
                  _row_spec(768), _row_spec(768), _row_spec(H),
                  _row_spec(128), _full_spec((MSG, 768)),
                  _full_spec((H, 896)), _full_spec((1, 768))],
        out_specs=[_row_spec(H), _row_spec(MSG), _row_spec(KEY),
                   _row_spec(KEY), _row_spec(768)],
        out_shape=[jax.ShapeDtypeStruct((N, H), jnp.float32),
                   jax.ShapeDtypeStruct((N, MSG), jnp.float32),
                   jax.ShapeDtypeStruct((N, KEY), jnp.float32),
                   jax.ShapeDtypeStruct((N, KEY), jnp.float32),
                   jax.ShapeDtypeStruct((N, 768), jnp.float32)],
        compiler_params=pltpu.CompilerParams(
            dimension_semantics=("parallel",)),
    )(acc, gf, gh, h, pf, wihct, wht, bhh)


def _gru_final(acc, gf, gh, h, wihct):
    return pl.pallas_call(
        _gru_final_body,
        grid=(GRID,),
        in_specs=[_acc_spec(acc.shape[0]),
                  _row_spec(768), _row_spec(768), _row_spec(H),
                  _full_spec((MSG, 768))],
        out_specs=_row_spec(H),
        out_shape=jax.ShapeDtypeStruct((N, H), jnp.float32),
        compiler_params=pltpu.CompilerParams(
            dimension_semantics=("parallel",)),
    )(acc, gf, gh, h, wihct)


NC, NS, L = 2, 16, 16
NW = NC * NS
CHUNK = 128
NCHUNK = 40
EPADT = NW * NCHUNK * CHUNK
RPT = NPAD // NS

_sc_mesh = plsc.VectorSubcoreMesh(core_axis_name="c", subcore_axis_name="s",
                                  num_cores=NC, num_subcores=NS)


def _gather_sq_body(src_hbm, dst_hbm, s_hbm, q_hbm, sg_hbm, qg_hbm,
                    src_v, dst_v, s_v, q_v):
    cid = lax.axis_index("c")
    sid = lax.axis_index("s")
    wid = cid * NS + sid
    pltpu.sync_copy(src_hbm.at[wid], src_v)
    pltpu.sync_copy(dst_hbm.at[wid], dst_v)

    def chunk_body(t, carry):
        row0 = (wid * NCHUNK + t) * CHUNK
        pltpu.sync_copy(s_hbm.at[src_v.at[t]], s_v)
        pltpu.sync_copy(q_hbm.at[dst_v.at[t]], q_v)
        pltpu.sync_copy(s_v, sg_hbm.at[pl.ds(row0, CHUNK)])
        pltpu.sync_copy(q_v, qg_hbm.at[pl.ds(row0, CHUNK)])
        return carry

    lax.fori_loop(0, NCHUNK, chunk_body, 0)


_gather_sq = pl.kernel(
    _gather_sq_body,
    out_type=(jax.ShapeDtypeStruct((EPADT, KEY), jnp.float32),
              jax.ShapeDtypeStruct((EPADT, KEY), jnp.float32)),
    mesh=_sc_mesh,
    compiler_params=pltpu.CompilerParams(use_tc_tiling_on_sc=False),
    scratch_types=[
        pltpu.VMEM((NCHUNK, CHUNK), jnp.int32),
        pltpu.VMEM((NCHUNK, CHUNK), jnp.int32),
        pltpu.VMEM((CHUNK, KEY), jnp.float32),
        pltpu.VMEM((CHUNK, KEY), jnp.float32),
    ],
)


BE = 2048


def _dot_exp_body(sg, qg, ee_o):
    p = sg[...] * qg[...]
    ee = jnp.exp(jnp.sum(p, axis=1) * (1.0 / KEY))
    ee_o[...] = jnp.broadcast_to(ee[:, None], (BE, L))


def _dot_exp(sg, qg):
    return pl.pallas_call(
        _dot_exp_body,
        grid=(EPADT // BE,),
        in_specs=[pl.BlockSpec((BE, KEY), lambda i: (i, 0)),
                  pl.BlockSpec((BE, KEY), lambda i: (i, 0))],
        out_specs=pl.BlockSpec((BE, L), lambda i: (i, 0)),
        out_shape=jax.ShapeDtypeStruct((EPADT, L), jnp.float32),
        compiler_params=pltpu.CompilerParams(
            dimension_semantics=("parallel",)),
    )(sg, qg)


def _scatter_acc_body(src_hbm, dst_hbm, v_hbm, ee_hbm, zeros_hbm, out_hbm,
                      src_v, dst_v, v_v, e_v, o_v, acc_sh):
    cid = lax.axis_index("c")
    sid = lax.axis_index("s")
    wid = cid * NS + sid
    pltpu.sync_copy(zeros_hbm.at[pl.ds(sid * RPT, RPT)],
                    acc_sh.at[pl.ds(sid * RPT, RPT)])
    pltpu.sync_copy(src_hbm.at[wid], src_v)
    pltpu.sync_copy(dst_hbm.at[wid], dst_v)
    plsc.subcore_barrier()

    iota = lax.iota(jnp.int32, L)
    unit0 = jnp.where(iota == 0, 1.0, 0.0).astype(jnp.float32)

    def chunk_body(t, carry):
        row0 = (wid * NCHUNK + t) * CHUNK
        pltpu.sync_copy(v_hbm.at[src_v.at[t]], v_v)
        pltpu.sync_copy(ee_hbm.at[pl.ds(row0, CHUNK)], e_v)

        def edge_body(ei, carry2):
            ee = e_v[ei, 0:L]
            o_v[ei, 0:L] = ee * unit0
            for k0 in range(MSG // L):
                o_v[ei, L + k0 * L:2 * L + k0 * L] = (
                    ee * v_v[ei, k0 * L:(k0 + 1) * L])
            return carry2

        lax.fori_loop(0, CHUNK, edge_body, 0)
        pltpu.sync_copy(o_v, acc_sh.at[dst_v.at[t]], add=True)
        return carry

    lax.fori_loop(0, NCHUNK, chunk_body, 0)
    plsc.subcore_barrier()
    pltpu.sync_copy(acc_sh.at[pl.ds(sid * RPT, RPT)],
                    out_hbm.at[cid, pl.ds(sid * RPT, RPT)])


_scatter_acc = pl.kernel(
    _scatter_acc_body,
    out_type=jax.ShapeDtypeStruct((NC, NPAD, ACC_W), jnp.float32),
    mesh=_sc_mesh,
    compiler_params=pltpu.CompilerParams(use_tc_tiling_on_sc=False),
    scratch_types=[
        pltpu.VMEM((NCHUNK, CHUNK), jnp.int32),
        pltpu.VMEM((NCHUNK, CHUNK), jnp.int32),
        pltpu.VMEM((CHUNK, MSG), jnp.float32),
        pltpu.VMEM((CHUNK, L), jnp.float32),
        pltpu.VMEM((CHUNK, ACC_W), jnp.float32),
        pltpu.VMEM_SHARED((NPAD, ACC_W), jnp.float32),
    ],
)


def _edge_pass(src_pad, dst_gpad, dst_spad, s, q, v, zeros):
    sg, qg = _gather_sq(src_pad, dst_gpad, s, q)
    eeb = _dot_exp(sg, qg)
    return _scatter_acc(src_pad, dst_spad, v, eeb, zeros)


def kernel(feat, h, edge_index, W_val, b_val, W_sign, b_sign, W_que, b_que,
           W_ih, b_ih, W_hh, b_hh):
    src = edge_index[0]
    dst = edge_index[1]
    wf = jnp.concatenate([W_val[:, :H], W_sign[:, :H], W_que[:, :H],
                          W_ih[:, :H]], axis=0)
    wh = jnp.concatenate([W_val[:, H:], W_sign[:, H:], W_que[:, H:],
                          W_hh], axis=0)
    wft = wf.T
    wht = wh.T
    wihct = W_ih[:, H:].T
    bp = jnp.concatenate([b_val, b_sign, b_que])[None]
    bih = b_ih[None]
    bhh = b_hh[None]

    pad = EPADT - E
    src_pad = jnp.concatenate(
        [src, jnp.zeros((pad,), jnp.int32)]).reshape(NW, NCHUNK, CHUNK)
    dst_gpad = jnp.concatenate(
        [dst, jnp.zeros((pad,), jnp.int32)]).reshape(NW, NCHUNK, CHUNK)
    dst_spad = jnp.concatenate(
        [dst, jnp.full((pad,), N, jnp.int32)]).reshape(NW, NCHUNK, CHUNK)
    zeros = jnp.zeros((NPAD, ACC_W), jnp.float32)

    pf, gf, v1, s1, q1, gh1 = _proj0(feat, h, wft, wht, bp, bih, bhh)
    acc1 = _edge_pass(src_pad, dst_gpad, dst_spad, s1, q1, v1, zeros)
    h1, v2, s2, q2, gh2 = _gru_proj(acc1, gf, gh1, h, pf, wihct, wht, bhh)
    acc2 = _edge_pass(src_pad, dst_gpad, dst_spad, s2, q2, v2, zeros)
    h2 = _gru_final(acc2, gf, gh2, h1, wihct)
    return (h2, h2)


# speedup vs baseline: 4.4348x; 1.1501x over previous
import functools

import jax
import jax.numpy as jnp
from jax import lax
from jax.experimental import pallas as pl
from jax.experimental.pallas import tpu as pltpu
from jax.experimental.pallas import tpu_sc as plsc

N = 10000
E = 160000
H = 256
MSG = 64
KEY = 32

NPAD = 10112
ACC_W = 80

BN = 1000
GRID = N // BN


def _proj0_body(feat, h, wft, wht, bp, bih, bhh,
                pf_o, gf_o, v_o, s_o, q_o, gh_o):
    a = jnp.dot(feat[...], wft[...], preferred_element_type=jnp.float32)
    b = jnp.dot(h[...], wht[...], preferred_element_type=jnp.float32)
    pf = a[:, :128] + bp[...]
    gf = a[:, 128:] + bih[...]
    p1 = pf + b[:, :128]
    pf_o[...] = pf
    gf_o[...] = gf
    v_o[...] = p1[:, :MSG]
    s_o[...] = p1[:, MSG:MSG + KEY]
    q_o[...] = p1[:, MSG + KEY:]
    gh_o[...] = b[:, 128:] + bhh[...]


def _gru_core(c, gf, gh, h, wihct):
    gi = gf + jnp.dot(c, wihct, preferred_element_type=jnp.float32)
    i_r, i_z, i_n = gi[:, :H], gi[:, H:2 * H], gi[:, 2 * H:]
    h_r, h_z, h_n = gh[:, :H], gh[:, H:2 * H], gh[:, 2 * H:]
    r = jax.nn.sigmoid(i_r + h_r)
    z = jax.nn.sigmoid(i_z + h_z)
    n = jnp.tanh(i_n + r * h_n)
    return (1.0 - z) * n + z * h


def _finalize_c(acc):
    a = jnp.sum(acc[...], axis=0)
    den = jnp.sum(a[:, :16], axis=1)
    num = a[:, 16:]
    return num * (1.0 / jnp.maximum(den, 1e-30))[:, None]


def _gru_proj_body(acc, gf, gh, h, pf, wihct, wht, bhh,
                   h1_o, v_o, s_o, q_o, gh_o):
    c = _finalize_c(acc)
    h1 = _gru_core(c, gf[...], gh[...], h[...], wihct[...])
    h1_o[...] = h1
    b2 = jnp.dot(h1, wht[...], preferred_element_type=jnp.float32)
    p2 = pf[...] + b2[:, :128]
    v_o[...] = p2[:, :MSG]
    s_o[...] = p2[:, MSG:MSG + KEY]
    q_o[...] = p2[:, MSG + KEY:]
    gh_o[...] = b2[:, 128:] + bhh[...]


def _gru_final_body(acc, gf, gh, h, wihct, h2_o):
    c = _finalize_c(acc)
    h2_o[...] = _gru_core(c, gf[...], gh[...], h[...], wihct[...])


def _row_spec(w):
    return pl.BlockSpec((BN, w), lambda i: (i, 0))


def _full_spec(shape):
    return pl.BlockSpec(shape, lambda i: tuple(0 for _ in shape))


def _proj0(feat, h, wft, wht, bp, bih, bhh):
    return pl.pallas_call(
        _proj0_body,
        grid=(GRID,),
        in_specs=[_row_spec(H), _row_spec(H), _full_spec((H, 896)),
                  _full_spec((H, 896)), _full_spec((1, 128)),
                  _full_spec((1, 768)), _full_spec((1, 768))],
        out_specs=[_row_spec(128), _row_spec(768), _row_spec(MSG),
                   _row_spec(KEY), _row_spec(KEY), _row_spec(768)],
        out_shape=[jax.ShapeDtypeStruct((N, 128), jnp.float32),
                   jax.ShapeDtypeStruct((N, 768), jnp.float32),
                   jax.ShapeDtypeStruct((N, MSG), jnp.float32),
                   jax.ShapeDtypeStruct((N, KEY), jnp.float32),
                   jax.ShapeDtypeStruct((N, KEY), jnp.float32),
                   jax.ShapeDtypeStruct((N, 768), jnp.float32)],
        compiler_params=pltpu.CompilerParams(
            dimension_semantics=("parallel",)),
    )(feat, h, wft, wht, bp, bih, bhh)


def _acc_spec(nacc):
    return pl.BlockSpec((nacc, BN, ACC_W), lambda i: (0, i, 0))


def _gru_proj(acc, gf, gh, h, pf, wihct, wht, bhh):
    return pl.pallas_call(
        _gru_proj_body,
        grid=(GRID,),
        in_specs=[_acc_spec(acc.shape[0]),
                  _row_spec(768), _row_spec(768), _row_spec(H),
                  _row_spec(128), _full_spec((MSG, 768)),
                  _full_spec((H, 896)), _full_spec((1, 768))],
        out_specs=[_row_spec(H), _row_spec(MSG), _row_spec(KEY),
                   _row_spec(KEY), _row_spec(768)],
        out_shape=[jax.ShapeDtypeStruct((N, H), jnp.float32),
                   jax.ShapeDtypeStruct((N, MSG), jnp.float32),
                   jax.ShapeDtypeStruct((N, KEY), jnp.float32),
                   jax.ShapeDtypeStruct((N, KEY), jnp.float32),
                   jax.ShapeDtypeStruct((N, 768), jnp.float32)],
        compiler_params=pltpu.CompilerParams(
            dimension_semantics=("parallel",)),
    )(acc, gf, gh, h, pf, wihct, wht, bhh)


def _gru_final(acc, gf, gh, h, wihct):
    return pl.pallas_call(
        _gru_final_body,
        grid=(GRID,),
        in_specs=[_acc_spec(acc.shape[0]),
                  _row_spec(768), _row_spec(768), _row_spec(H),
                  _full_spec((MSG, 768))],
        out_specs=_row_spec(H),
        out_shape=jax.ShapeDtypeStruct((N, H), jnp.float32),
        compiler_params=pltpu.CompilerParams(
            dimension_semantics=("parallel",)),
    )(acc, gf, gh, h, wihct)


NC, NS, L = 2, 16, 16
NW = NC * NS
CHUNK = 128
NCHUNK = 40
EPADT = NW * NCHUNK * CHUNK
RPT = NPAD // NS

_sc_mesh = plsc.VectorSubcoreMesh(core_axis_name="c", subcore_axis_name="s",
                                  num_cores=NC, num_subcores=NS)


def _gather_sqv_body(src_hbm, dst_hbm, s_hbm, q_hbm, v_hbm,
                     sg_hbm, qg_hbm, vg_hbm,
                     src_v, dst_v, s_v, q_v, v_v, gsem, wsem):
    cid = lax.axis_index("c")
    sid = lax.axis_index("s")
    wid = cid * NS + sid
    pltpu.sync_copy(src_hbm.at[wid], src_v)
    pltpu.sync_copy(dst_hbm.at[wid], dst_v)

    def start_gather(t, b):
        pltpu.async_copy(s_hbm.at[src_v.at[t]], s_v.at[b], gsem)
        pltpu.async_copy(q_hbm.at[dst_v.at[t]], q_v.at[b], gsem)
        pltpu.async_copy(v_hbm.at[src_v.at[t]], v_v.at[b], gsem)

    def wait_gather(t, b):
        pltpu.make_async_copy(s_hbm.at[src_v.at[t]], s_v.at[b], gsem).wait()
        pltpu.make_async_copy(q_hbm.at[dst_v.at[t]], q_v.at[b], gsem).wait()
        pltpu.make_async_copy(v_hbm.at[src_v.at[t]], v_v.at[b], gsem).wait()

    def start_write(t, b):
        row0 = (wid * NCHUNK + t) * CHUNK
        pltpu.async_copy(s_v.at[b], sg_hbm.at[pl.ds(row0, CHUNK)], wsem)
        pltpu.async_copy(q_v.at[b], qg_hbm.at[pl.ds(row0, CHUNK)], wsem)
        pltpu.async_copy(v_v.at[b], vg_hbm.at[pl.ds(row0, CHUNK)], wsem)

    def wait_write(t, b):
        row0 = (wid * NCHUNK + t) * CHUNK
        pltpu.make_async_copy(s_v.at[b], sg_hbm.at[pl.ds(row0, CHUNK)], wsem).wait()
        pltpu.make_async_copy(q_v.at[b], qg_hbm.at[pl.ds(row0, CHUNK)], wsem).wait()
        pltpu.make_async_copy(v_v.at[b], vg_hbm.at[pl.ds(row0, CHUNK)], wsem).wait()

    start_gather(0, 0)

    def pair_body(tt, carry):
        for b in range(2):
            t = 2 * tt + b
            nb = 1 - b
            wait_gather(t, b)

            @pl.when(t + 1 < NCHUNK)
            def _():
                @pl.when(t >= 1)
                def _():
                    wait_write(t - 1, nb)
                start_gather(t + 1, nb)

            start_write(t, b)
        return carry

    lax.fori_loop(0, NCHUNK // 2, pair_body, 0)
    wait_write(NCHUNK - 2, 0)
    wait_write(NCHUNK - 1, 1)


_gather_sqv = pl.kernel(
    _gather_sqv_body,
    out_type=(jax.ShapeDtypeStruct((EPADT, KEY), jnp.float32),
              jax.ShapeDtypeStruct((EPADT, KEY), jnp.float32),
              jax.ShapeDtypeStruct((EPADT, MSG), jnp.float32)),
    mesh=_sc_mesh,
    compiler_params=pltpu.CompilerParams(use_tc_tiling_on_sc=False),
    scratch_types=[
        pltpu.VMEM((NCHUNK, CHUNK), jnp.int32),
        pltpu.VMEM((NCHUNK, CHUNK), jnp.int32),
        pltpu.VMEM((2, CHUNK, KEY), jnp.float32),
        pltpu.VMEM((2, CHUNK, KEY), jnp.float32),
        pltpu.VMEM((2, CHUNK, MSG), jnp.float32),
        pltpu.SemaphoreType.DMA,
        pltpu.SemaphoreType.DMA,
    ],
)


BE = 2048


def _build_m_body(sg, qg, vg, m_o):
    p = sg[...] * qg[...]
    ee = jnp.exp(jnp.sum(p, axis=1) * (1.0 / KEY))[:, None]
    m_o[:, 0:L] = jnp.concatenate(
        [ee, jnp.zeros((BE, L - 1), jnp.float32)], axis=1)
    m_o[:, L:] = ee * vg[...]


def _build_m(sg, qg, vg):
    return pl.pallas_call(
        _build_m_body,
        grid=(EPADT // BE,),
        in_specs=[pl.BlockSpec((BE, KEY), lambda i: (i, 0)),
                  pl.BlockSpec((BE, KEY), lambda i: (i, 0)),
                  pl.BlockSpec((BE, MSG), lambda i: (i, 0))],
        out_specs=pl.BlockSpec((BE, ACC_W), lambda i: (i, 0)),
        out_shape=jax.ShapeDtypeStruct((EPADT, ACC_W), jnp.float32),
        compiler_params=pltpu.CompilerParams(
            dimension_semantics=("parallel",)),
    )(sg, qg, vg)


def _scatter_acc_body(dst_hbm, m_hbm, zeros_hbm, out_hbm,
                      dst_v, o_v, lsem, acc_sh):
    cid = lax.axis_index("c")
    sid = lax.axis_index("s")
    wid = cid * NS + sid
    pltpu.sync_copy(zeros_hbm.at[pl.ds(sid * RPT, RPT)],
                    acc_sh.at[pl.ds(sid * RPT, RPT)])
    pltpu.sync_copy(dst_hbm.at[wid], dst_v)
    plsc.subcore_barrier()

    def load_m(t, b):
        row0 = (wid * NCHUNK + t) * CHUNK
        return pltpu.make_async_copy(m_hbm.at[pl.ds(row0, CHUNK)],
                                     o_v.at[b], lsem)

    def start_load(t, b):
        row0 = (wid * NCHUNK + t) * CHUNK
        pltpu.async_copy(m_hbm.at[pl.ds(row0, CHUNK)], o_v.at[b], lsem)

    start_load(0, 0)

    def pair_body(tt, carry):
        for b in range(2):
            t = 2 * tt + b
            load_m(t, b).wait()

            @pl.when(t + 1 < NCHUNK)
            def _():
                start_load(t + 1, 1 - b)

            pltpu.sync_copy(o_v.at[b], acc_sh.at[dst_v.at[t]], add=True)
        return carry

    lax.fori_loop(0, NCHUNK // 2, pair_body, 0)
    plsc.subcore_barrier()
    pltpu.sync_copy(acc_sh.at[pl.ds(sid * RPT, RPT)],
                    out_hbm.at[cid, pl.ds(sid * RPT, RPT)])


_scatter_acc = pl.kernel(
    _scatter_acc_body,
    out_type=jax.ShapeDtypeStruct((NC, NPAD, ACC_W), jnp.float32),
    mesh=_sc_mesh,
    compiler_params=pltpu.CompilerParams(use_tc_tiling_on_sc=False),
    scratch_types=[
        pltpu.VMEM((NCHUNK, CHUNK), jnp.int32),
        pltpu.VMEM((2, CHUNK, ACC_W), jnp.float32),
        pltpu.SemaphoreType.DMA,
        pltpu.VMEM_SHARED((NPAD, ACC_W), jnp.float32),
    ],
)


def _edge_pass(src_pad, dst_gpad, dst_spad, s, q, v, zeros):
    sg, qg, vg = _gather_sqv(src_pad, dst_gpad, s, q, v)
    m = _build_m(sg, qg, vg)
    return _scatter_acc(dst_spad, m, zeros)


def kernel(feat, h, edge_index, W_val, b_val, W_sign, b_sign, W_que, b_que,
           W_ih, b_ih, W_hh, b_hh):
    src = edge_index[0]
    dst = edge_index[1]
    wf = jnp.concatenate([W_val[:, :H], W_sign[:, :H], W_que[:, :H],
                          W_ih[:, :H]], axis=0)
    wh = jnp.concatenate([W_val[:, H:], W_sign[:, H:], W_que[:, H:],
                          W_hh], axis=0)
    wft = wf.T
    wht = wh.T
    wihct = W_ih[:, H:].T
    bp = jnp.concatenate([b_val, b_sign, b_que])[None]
    bih = b_ih[None]
    bhh = b_hh[None]

    pad = EPADT - E
    src_pad = jnp.concatenate(
        [src, jnp.zeros((pad,), jnp.int32)]).reshape(NW, NCHUNK, CHUNK)
    dst_gpad = jnp.concatenate(
        [dst, jnp.zeros((pad,), jnp.int32)]).reshape(NW, NCHUNK, CHUNK)
    dst_spad = jnp.concatenate(
        [dst, jnp.full((pad,), N, jnp.int32)]).reshape(NW, NCHUNK, CHUNK)
    zeros = jnp.zeros((NPAD, ACC_W), jnp.float32)

    pf, gf, v1, s1, q1, gh1 = _proj0(feat, h, wft, wht, bp, bih, bhh)
    acc1 = _edge_pass(src_pad, dst_gpad, dst_spad, s1, q1, v1, zeros)
    h1, v2, s2, q2, gh2 = _gru_proj(acc1, gf, gh1, h, pf, wihct, wht, bhh)
    acc2 = _edge_pass(src_pad, dst_gpad, dst_spad, s2, q2, v2, zeros)
    h2 = _gru_final(acc2, gf, gh2, h1, wihct)
    return (h2, h2)


# speedup vs baseline: 6.9668x; 1.5709x over previous
import functools

import jax
import jax.numpy as jnp
from jax import lax
from jax.experimental import pallas as pl
from jax.experimental.pallas import tpu as pltpu
from jax.experimental.pallas import tpu_sc as plsc

N = 10000
E = 160000
H = 256
MSG = 64
KEY = 32

NPAD = 10112
ACC_W = 128

BN = 1000
GRID = N // BN


def _proj0_body(feat, h, wft, wht, bp, bih, bhh,
                pf_o, gf_o, v_o, s_o, q_o, gh_o):
    a = jnp.dot(feat[...], wft[...], preferred_element_type=jnp.float32)
    b = jnp.dot(h[...], wht[...], preferred_element_type=jnp.float32)
    pf = a[:, :128] + bp[...]
    gf = a[:, 128:] + bih[...]
    p1 = pf + b[:, :128]
    pf_o[...] = pf
    gf_o[...] = gf
    v_o[...] = p1[:, :MSG]
    s_o[...] = p1[:, MSG:MSG + KEY]
    q_o[...] = p1[:, MSG + KEY:]
    gh_o[...] = b[:, 128:] + bhh[...]


def _gru_core(c, gf, gh, h, wihct):
    gi = gf + jnp.dot(c, wihct, preferred_element_type=jnp.float32)
    i_r, i_z, i_n = gi[:, :H], gi[:, H:2 * H], gi[:, 2 * H:]
    h_r, h_z, h_n = gh[:, :H], gh[:, H:2 * H], gh[:, 2 * H:]
    r = jax.nn.sigmoid(i_r + h_r)
    z = jax.nn.sigmoid(i_z + h_z)
    n = jnp.tanh(i_n + r * h_n)
    return (1.0 - z) * n + z * h


def _finalize_c(acc):
    a = jnp.sum(acc[...], axis=0)
    den = jnp.sum(a[:, :16], axis=1)
    num = a[:, 16:16 + MSG]
    return num * (1.0 / jnp.maximum(den, 1e-30))[:, None]


def _gru_proj_body(acc, gf, gh, h, pf, wihct, wht, bhh,
                   h1_o, v_o, s_o, q_o, gh_o):
    c = _finalize_c(acc)
    h1 = _gru_core(c, gf[...], gh[...], h[...], wihct[...])
    h1_o[...] = h1
    b2 = jnp.dot(h1, wht[...], preferred_element_type=jnp.float32)
    p2 = pf[...] + b2[:, :128]
    v_o[...] = p2[:, :MSG]
    s_o[...] = p2[:, MSG:MSG + KEY]
    q_o[...] = p2[:, MSG + KEY:]
    gh_o[...] = b2[:, 128:] + bhh[...]


def _gru_final_body(acc, gf, gh, h, wihct, h2_o):
    c = _finalize_c(acc)
    h2_o[...] = _gru_core(c, gf[...], gh[...], h[...], wihct[...])


def _row_spec(w):
    return pl.BlockSpec((BN, w), lambda i: (i, 0))


def _full_spec(shape):
    return pl.BlockSpec(shape, lambda i: tuple(0 for _ in shape))


def _proj0(feat, h, wft, wht, bp, bih, bhh):
    return pl.pallas_call(
        _proj0_body,
        grid=(GRID,),
        in_specs=[_row_spec(H), _row_spec(H), _full_spec((H, 896)),
                  _full_spec((H, 896)), _full_spec((1, 128)),
                  _full_spec((1, 768)), _full_spec((1, 768))],
        out_specs=[_row_spec(128), _row_spec(768), _row_spec(MSG),
                   _row_spec(KEY), _row_spec(KEY), _row_spec(768)],
        out_shape=[jax.ShapeDtypeStruct((N, 128), jnp.float32),
                   jax.ShapeDtypeStruct((N, 768), jnp.float32),
                   jax.ShapeDtypeStruct((N, MSG), jnp.float32),
                   jax.ShapeDtypeStruct((N, KEY), jnp.float32),
                   jax.ShapeDtypeStruct((N, KEY), jnp.float32),
                   jax.ShapeDtypeStruct((N, 768), jnp.float32)],
        compiler_params=pltpu.CompilerParams(
            dimension_semantics=("parallel",)),
    )(feat, h, wft, wht, bp, bih, bhh)


def _acc_spec(nacc):
    return pl.BlockSpec((nacc, BN, ACC_W), lambda i: (0, i, 0))


def _gru_proj(acc, gf, gh, h, pf, wihct, wht, bhh):
    return pl.pallas_call(
        _gru_proj_body,
        grid=(GRID,),
        in_specs=[_acc_spec(acc.shape[0]),
                  _row_spec(768), _row_spec(768), _row_spec(H),
                  _row_spec(128), _full_spec((MSG, 768)),
                  _full_spec((H, 896)), _full_spec((1, 768))],
        out_specs=[_row_spec(H), _row_spec(MSG), _row_spec(KEY),
                   _row_spec(KEY), _row_spec(768)],
        out_shape=[jax.ShapeDtypeStruct((N, H), jnp.float32),
                   jax.ShapeDtypeStruct((N, MSG), jnp.float32),
                   jax.ShapeDtypeStruct((N, KEY), jnp.float32),
                   jax.ShapeDtypeStruct((N, KEY), jnp.float32),
                   jax.ShapeDtypeStruct((N, 768), jnp.float32)],
        compiler_params=pltpu.CompilerParams(
            dimension_semantics=("parallel",)),
    )(acc, gf, gh, h, pf, wihct, wht, bhh)


def _gru_final(acc, gf, gh, h, wihct):
    return pl.pallas_call(
        _gru_final_body,
        grid=(GRID,),
        in_specs=[_acc_spec(acc.shape[0]),
                  _row_spec(768), _row_spec(768), _row_spec(H),
                  _full_spec((MSG, 768))],
        out_specs=_row_spec(H),
        out_shape=jax.ShapeDtypeStruct((N, H), jnp.float32),
        compiler_params=pltpu.CompilerParams(
            dimension_semantics=("parallel",)),
    )(acc, gf, gh, h, wihct)


NC, NS, L = 2, 16, 16
NW = NC * NS
CHUNK = 128
NCHUNK = 40
EPADT = NW * NCHUNK * CHUNK
RPT = NPAD // NS

_sc_mesh = plsc.VectorSubcoreMesh(core_axis_name="c", subcore_axis_name="s",
                                  num_cores=NC, num_subcores=NS)


def _gather_sqv_body(src_hbm, dst_hbm, s_hbm, q_hbm, v_hbm,
                     g_hbm,
                     src_v, dst_v, s_v, q_v, v_v, gsem, wsem):
    cid = lax.axis_index("c")
    sid = lax.axis_index("s")
    wid = cid * NS + sid
    pltpu.sync_copy(src_hbm.at[wid], src_v)
    pltpu.sync_copy(dst_hbm.at[wid], dst_v)

    def start_gather(t, b):
        pltpu.async_copy(s_hbm.at[src_v.at[t]], s_v.at[b], gsem)
        pltpu.async_copy(q_hbm.at[dst_v.at[t]], q_v.at[b], gsem)
        pltpu.async_copy(v_hbm.at[src_v.at[t]], v_v.at[b], gsem)

    def wait_gather(t, b):
        pltpu.make_async_copy(s_hbm.at[src_v.at[t]], s_v.at[b], gsem).wait()
        pltpu.make_async_copy(q_hbm.at[dst_v.at[t]], q_v.at[b], gsem).wait()
        pltpu.make_async_copy(v_hbm.at[src_v.at[t]], v_v.at[b], gsem).wait()

    def _write_descs(t, b):
        row0 = (wid * NCHUNK + t) * CHUNK
        rows = pl.ds(row0, CHUNK)
        return ((s_v.at[b], g_hbm.at[rows, pl.ds(0, KEY)]),
                (q_v.at[b], g_hbm.at[rows, pl.ds(KEY, KEY)]),
                (v_v.at[b], g_hbm.at[rows, pl.ds(2 * KEY, MSG)]))

    def start_write(t, b):
        for src_r, dst_r in _write_descs(t, b):
            pltpu.async_copy(src_r, dst_r, wsem)

    def wait_write(t, b):
        for src_r, dst_r in _write_descs(t, b):
            pltpu.make_async_copy(src_r, dst_r, wsem).wait()

    start_gather(0, 0)

    def pair_body(tt, carry):
        for b in range(2):
            t = 2 * tt + b
            nb = 1 - b
            wait_gather(t, b)

            @pl.when(t + 1 < NCHUNK)
            def _():
                @pl.when(t >= 1)
                def _():
                    wait_write(t - 1, nb)
                start_gather(t + 1, nb)

            start_write(t, b)
        return carry

    lax.fori_loop(0, NCHUNK // 2, pair_body, 0)
    wait_write(NCHUNK - 2, 0)
    wait_write(NCHUNK - 1, 1)


_gather_sqv = pl.kernel(
    _gather_sqv_body,
    out_type=jax.ShapeDtypeStruct((EPADT, 2 * KEY + MSG), jnp.float32),
    mesh=_sc_mesh,
    compiler_params=pltpu.CompilerParams(use_tc_tiling_on_sc=False),
    scratch_types=[
        pltpu.VMEM((NCHUNK, CHUNK), jnp.int32),
        pltpu.VMEM((NCHUNK, CHUNK), jnp.int32),
        pltpu.VMEM((2, CHUNK, KEY), jnp.float32),
        pltpu.VMEM((2, CHUNK, KEY), jnp.float32),
        pltpu.VMEM((2, CHUNK, MSG), jnp.float32),
        pltpu.SemaphoreType.DMA,
        pltpu.SemaphoreType.DMA,
    ],
)


BE = 2048


def _build_m_body(g, m_o):
    gb = g[...]
    p = gb[:, 0:KEY] * gb[:, KEY:2 * KEY]
    ee = jnp.exp(jnp.sum(p, axis=1) * (1.0 / KEY))[:, None]
    m_o[...] = jnp.concatenate(
        [ee, jnp.zeros((BE, L - 1), jnp.float32),
         ee * gb[:, 2 * KEY:],
         jnp.zeros((BE, ACC_W - L - MSG), jnp.float32)], axis=1)


def _build_m(g):
    return pl.pallas_call(
        _build_m_body,
        grid=(EPADT // BE,),
        in_specs=[pl.BlockSpec((BE, 2 * KEY + MSG), lambda i: (i, 0))],
        out_specs=pl.BlockSpec((BE, ACC_W), lambda i: (i, 0)),
        out_shape=jax.ShapeDtypeStruct((EPADT, ACC_W), jnp.float32),
        compiler_params=pltpu.CompilerParams(
            dimension_semantics=("parallel",)),
    )(g)


def _scatter_acc_body(dst_hbm, m_hbm, zeros_hbm, out_hbm,
                      dst_v, o_v, lsem, acc_sh):
    cid = lax.axis_index("c")
    sid = lax.axis_index("s")
    wid = cid * NS + sid
    pltpu.sync_copy(zeros_hbm.at[pl.ds(sid * RPT, RPT)],
                    acc_sh.at[pl.ds(sid * RPT, RPT)])
    pltpu.sync_copy(dst_hbm.at[wid], dst_v)
    plsc.subcore_barrier()

    def load_m(t, b):
        row0 = (wid * NCHUNK + t) * CHUNK
        return pltpu.make_async_copy(m_hbm.at[pl.ds(row0, CHUNK)],
                                     o_v.at[b], lsem)

    def start_load(t, b):
        row0 = (wid * NCHUNK + t) * CHUNK
        pltpu.async_copy(m_hbm.at[pl.ds(row0, CHUNK)], o_v.at[b], lsem)

    start_load(0, 0)

    def pair_body(tt, carry):
        for b in range(2):
            t = 2 * tt + b
            load_m(t, b).wait()

            @pl.when(t + 1 < NCHUNK)
            def _():
                start_load(t + 1, 1 - b)

            pltpu.sync_copy(o_v.at[b], acc_sh.at[dst_v.at[t]], add=True)
        return carry

    lax.fori_loop(0, NCHUNK // 2, pair_body, 0)
    plsc.subcore_barrier()
    pltpu.sync_copy(acc_sh.at[pl.ds(sid * RPT, RPT)],
                    out_hbm.at[cid, pl.ds(sid * RPT, RPT)])


_scatter_acc = pl.kernel(
    _scatter_acc_body,
    out_type=jax.ShapeDtypeStruct((NC, NPAD, ACC_W), jnp.float32),
    mesh=_sc_mesh,
    compiler_params=pltpu.CompilerParams(use_tc_tiling_on_sc=False),
    scratch_types=[
        pltpu.VMEM((NCHUNK, CHUNK), jnp.int32),
        pltpu.VMEM((2, CHUNK, ACC_W), jnp.float32),
        pltpu.SemaphoreType.DMA,
        pltpu.VMEM_SHARED((NPAD, ACC_W), jnp.float32),
    ],
)


def _edge_pass(src_pad, dst_gpad, dst_spad, s, q, v, zeros):
    g = _gather_sqv(src_pad, dst_gpad, s, q, v)
    m = _build_m(g)
    return _scatter_acc(dst_spad, m, zeros)


def kernel(feat, h, edge_index, W_val, b_val, W_sign, b_sign, W_que, b_que,
           W_ih, b_ih, W_hh, b_hh):
    src = edge_index[0]
    dst = edge_index[1]
    wf = jnp.concatenate([W_val[:, :H], W_sign[:, :H], W_que[:, :H],
                          W_ih[:, :H]], axis=0)
    wh = jnp.concatenate([W_val[:, H:], W_sign[:, H:], W_que[:, H:],
                          W_hh], axis=0)
    wft = wf.T
    wht = wh.T
    wihct = W_ih[:, H:].T
    bp = jnp.concatenate([b_val, b_sign, b_que])[None]
    bih = b_ih[None]
    bhh = b_hh[None]

    pad = EPADT - E
    src_pad = jnp.concatenate(
        [src, jnp.zeros((pad,), jnp.int32)]).reshape(NW, NCHUNK, CHUNK)
    dst_gpad = jnp.concatenate(
        [dst, jnp.zeros((pad,), jnp.int32)]).reshape(NW, NCHUNK, CHUNK)
    dst_spad = jnp.concatenate(
        [dst, jnp.full((pad,), N, jnp.int32)]).reshape(NW, NCHUNK, CHUNK)
    zeros = jnp.zeros((NPAD, ACC_W), jnp.float32)

    pf, gf, v1, s1, q1, gh1 = _proj0(feat, h, wft, wht, bp, bih, bhh)
    acc1 = _edge_pass(src_pad, dst_gpad, dst_spad, s1, q1, v1, zeros)
    h1, v2, s2, q2, gh2 = _gru_proj(acc1, gf, gh1, h, pf, wihct, wht, bhh)
    acc2 = _edge_pass(src_pad, dst_gpad, dst_spad, s2, q2, v2, zeros)
    h2 = _gru_final(acc2, gf, gh2, h1, wihct)
    return (h2, h2)
